# Initial kernel scaffold; baseline (speedup 1.0000x reference)
import jax
import jax.numpy as jnp
from jax.experimental import pallas as pl


def kernel(input, embedding):
    raise NotImplementedError("write your pallas kernel here")


# speedup vs baseline: 2.3501x; 2.3501x over previous
import functools

import jax
import jax.numpy as jnp
from jax import lax
from jax.experimental import pallas as pl
from jax.experimental.pallas import tpu as pltpu


def _vq_body(n_total, d_dim, k_dim, x_ref, emb_ref, qst_ref, enc_ref,
             loss_ref, perp_ref, counts_ref, sse_ref):
    i = pl.program_id(0)
    nb = pl.num_programs(0)

    @pl.when(i == 0)
    def _init():
        counts_ref[...] = jnp.zeros_like(counts_ref)
        sse_ref[0] = 0.0

    x = x_ref[...]
    emb = emb_ref[...]
    x2 = jnp.sum(x * x, axis=1, keepdims=True)
    e2 = jnp.sum(emb * emb, axis=1)
    xe = lax.dot_general(x, emb, (((1,), (1,)), ((), ())),
                         preferred_element_type=jnp.float32)
    d = x2 - 2.0 * xe + e2[None, :]
    dmin = jnp.min(d, axis=1, keepdims=True)
    kiota = lax.broadcasted_iota(jnp.int32, d.shape, 1)
    idx = jnp.min(jnp.where(d == dmin, kiota, k_dim), axis=1, keepdims=True)
    enc = (kiota == idx).astype(jnp.float32)
    enc_ref[...] = enc
    q = lax.dot_general(enc, emb, (((1,), (0,)), ((), ())),
                        preferred_element_type=jnp.float32)
    qst_ref[...] = x + (q - x)

    counts_ref[...] += jnp.sum(enc, axis=0, keepdims=True)
    sse_ref[0] += jnp.sum((q - x) * (q - x))

    @pl.when(i == nb - 1)
    def _fini():
        loss_ref[0] = sse_ref[0] / (n_total * d_dim)
        avg = counts_ref[...] * (1.0 / n_total)
        perp_ref[0] = jnp.exp(-jnp.sum(avg * jnp.log(avg + 1e-10)))


def kernel(input, embedding):
    b, t, ld = input.shape
    l_dim, k_dim, d_dim = embedding.shape
    n = b * t * (ld // d_dim)
    flat = input.reshape(n, d_dim)
    emb = embedding.reshape(k_dim, d_dim)

    bn = 512
    grid = (n // bn,)

    qst, enc, loss, perp = pl.pallas_call(
        functools.partial(_vq_body, n, d_dim, k_dim),
        grid=grid,
        in_specs=[
            pl.BlockSpec((bn, d_dim), lambda i: (i, 0)),
            pl.BlockSpec((k_dim, d_dim), lambda i: (0, 0)),
        ],
        out_specs=[
            pl.BlockSpec((bn, d_dim), lambda i: (i, 0)),
            pl.BlockSpec((bn, k_dim), lambda i: (i, 0)),
            pl.BlockSpec(memory_space=pltpu.SMEM),
            pl.BlockSpec(memory_space=pltpu.SMEM),
        ],
        out_shape=[
            jax.ShapeDtypeStruct((n, d_dim), jnp.float32),
            jax.ShapeDtypeStruct((n, k_dim), jnp.float32),
            jax.ShapeDtypeStruct((1,), jnp.float32),
            jax.ShapeDtypeStruct((1,), jnp.float32),
        ],
        scratch_shapes=[
            pltpu.VMEM((1, k_dim), jnp.float32),
            pltpu.SMEM((1,), jnp.float32),
        ],
    )(flat, emb)

    quantized_st = qst.reshape(input.shape)
    encodings = enc.reshape(n, l_dim, k_dim)
    return quantized_st, encodings, loss.reshape(()), perp.reshape(())


# speedup vs baseline: 2.4154x; 1.0278x over previous
import functools

import jax
import jax.numpy as jnp
from jax import lax
from jax.experimental import pallas as pl
from jax.experimental.pallas import tpu as pltpu


def _vq_body(n_total, d_dim, k_dim, x_ref, emb_ref, qst_ref, enc_ref,
             loss_ref, perp_ref, counts_ref, sse_ref):
    i = pl.program_id(0)
    nb = pl.num_programs(0)

    @pl.when(i == 0)
    def _init():
        counts_ref[...] = jnp.zeros_like(counts_ref)
        sse_ref[0] = 0.0

    x = x_ref[...]
    emb = emb_ref[...]
    x2 = jnp.sum(x * x, axis=1, keepdims=True)
    e2 = jnp.sum(emb * emb, axis=1)
    xe = lax.dot_general(x, emb, (((1,), (1,)), ((), ())),
                         preferred_element_type=jnp.float32)
    d = x2 - 2.0 * xe + e2[None, :]
    dmin = jnp.min(d, axis=1, keepdims=True)
    kiota = lax.broadcasted_iota(jnp.int32, d.shape, 1)
    kbits = lax.bitcast_convert_type(kiota | jnp.int32(0x3F800000),
                                     jnp.float32)
    sentinel = lax.bitcast_convert_type(jnp.int32(k_dim | 0x3F800000),
                                        jnp.float32)
    idx = jnp.min(jnp.where(d == dmin, kbits, sentinel), axis=1,
                  keepdims=True)
    enc = (kbits == idx).astype(jnp.float32)
    enc_ref[...] = enc
    q = lax.dot_general(enc, emb, (((1,), (0,)), ((), ())),
                        preferred_element_type=jnp.float32)
    qst_ref[...] = x + (q - x)

    counts_ref[...] += jnp.sum(enc, axis=0, keepdims=True)
    sse_ref[0] += jnp.sum((q - x) * (q - x))

    @pl.when(i == nb - 1)
    def _fini():
        loss_ref[0] = sse_ref[0] / (n_total * d_dim)
        avg = counts_ref[...] * (1.0 / n_total)
        perp_ref[0] = jnp.exp(-jnp.sum(avg * jnp.log(avg + 1e-10)))


def kernel(input, embedding):
    b, t, ld = input.shape
    l_dim, k_dim, d_dim = embedding.shape
    n = b * t * (ld // d_dim)
    flat = input.reshape(n, d_dim)
    emb = embedding.reshape(k_dim, d_dim)

    bn = 512
    grid = (n // bn,)

    qst, enc, loss, perp = pl.pallas_call(
        functools.partial(_vq_body, n, d_dim, k_dim),
        grid=grid,
        in_specs=[
            pl.BlockSpec((bn, d_dim), lambda i: (i, 0)),
            pl.BlockSpec((k_dim, d_dim), lambda i: (0, 0)),
        ],
        out_specs=[
            pl.BlockSpec((bn, d_dim), lambda i: (i, 0)),
            pl.BlockSpec((bn, k_dim), lambda i: (i, 0)),
            pl.BlockSpec(memory_space=pltpu.SMEM),
            pl.BlockSpec(memory_space=pltpu.SMEM),
        ],
        out_shape=[
            jax.ShapeDtypeStruct((n, d_dim), jnp.float32),
            jax.ShapeDtypeStruct((n, k_dim), jnp.float32),
            jax.ShapeDtypeStruct((1,), jnp.float32),
            jax.ShapeDtypeStruct((1,), jnp.float32),
        ],
        scratch_shapes=[
            pltpu.VMEM((1, k_dim), jnp.float32),
            pltpu.SMEM((1,), jnp.float32),
        ],
    )(flat, emb)

    quantized_st = qst.reshape(input.shape)
    encodings = enc.reshape(n, l_dim, k_dim)
    return quantized_st, encodings, loss.reshape(()), perp.reshape(())


# speedup vs baseline: 5.1372x; 2.1268x over previous
import functools

import jax
import jax.numpy as jnp
from jax import lax
from jax.experimental import pallas as pl
from jax.experimental.pallas import tpu as pltpu


def _vq_body(n_total, d_dim, k_dim, x_ref, emb_ref, qst_ref, enc_ref,
             loss_ref, perp_ref, counts_ref, sse_ref):
    i = pl.program_id(0)
    nb = pl.num_programs(0)
    bn = x_ref.shape[1]

    @pl.when(i == 0)
    def _init():
        counts_ref[...] = jnp.zeros_like(counts_ref)
        sse_ref[0] = 0.0

    x = x_ref[...].reshape(bn, d_dim)
    emb = emb_ref[...].reshape(k_dim, d_dim)
    x2 = jnp.sum(x * x, axis=1, keepdims=True)
    e2 = jnp.sum(emb * emb, axis=1)
    xe = lax.dot_general(x, emb, (((1,), (1,)), ((), ())),
                         preferred_element_type=jnp.float32)
    d = x2 - 2.0 * xe + e2[None, :]
    dmin = jnp.min(d, axis=1, keepdims=True)
    kiota = lax.broadcasted_iota(jnp.int32, d.shape, 1)
    kbits = lax.bitcast_convert_type(kiota | jnp.int32(0x3F800000),
                                     jnp.float32)
    sentinel = lax.bitcast_convert_type(jnp.int32(k_dim | 0x3F800000),
                                        jnp.float32)
    idx = jnp.min(jnp.where(d == dmin, kbits, sentinel), axis=1,
                  keepdims=True)
    enc = (kbits == idx).astype(jnp.float32)
    enc_ref[...] = enc.reshape(bn, 1, k_dim)
    q = lax.dot_general(enc, emb, (((1,), (0,)), ((), ())),
                        preferred_element_type=jnp.float32)
    qst_ref[...] = (x + (q - x)).reshape(1, bn, d_dim)

    counts_ref[...] += jnp.sum(enc, axis=0, keepdims=True)
    sse_ref[0] += jnp.sum((q - x) * (q - x))

    @pl.when(i == nb - 1)
    def _fini():
        loss_ref[0] = sse_ref[0] / (n_total * d_dim)
        avg = counts_ref[...] * (1.0 / n_total)
        perp_ref[0] = jnp.exp(-jnp.sum(avg * jnp.log(avg + 1e-10)))


def kernel(input, embedding):
    b, t, ld = input.shape
    l_dim, k_dim, d_dim = embedding.shape
    n = b * t * (ld // d_dim)
    bn = n // b
    grid = (b,)

    qst, enc, loss, perp = pl.pallas_call(
        functools.partial(_vq_body, n, d_dim, k_dim),
        grid=grid,
        in_specs=[
            pl.BlockSpec((1, t, ld), lambda i: (i, 0, 0)),
            pl.BlockSpec((l_dim, k_dim, d_dim), lambda i: (0, 0, 0)),
        ],
        out_specs=[
            pl.BlockSpec((1, t, ld), lambda i: (i, 0, 0)),
            pl.BlockSpec((bn, 1, k_dim), lambda i: (i, 0, 0)),
            pl.BlockSpec(memory_space=pltpu.SMEM),
            pl.BlockSpec(memory_space=pltpu.SMEM),
        ],
        out_shape=[
            jax.ShapeDtypeStruct((b, t, ld), jnp.float32),
            jax.ShapeDtypeStruct((n, l_dim, k_dim), jnp.float32),
            jax.ShapeDtypeStruct((1,), jnp.float32),
            jax.ShapeDtypeStruct((1,), jnp.float32),
        ],
        scratch_shapes=[
            pltpu.VMEM((1, k_dim), jnp.float32),
            pltpu.SMEM((1,), jnp.float32),
        ],
    )(input, embedding)

    return qst, enc, loss.reshape(()), perp.reshape(())


# speedup vs baseline: 5.6094x; 1.0919x over previous
import functools

import jax
import jax.numpy as jnp
from jax import lax
from jax.experimental import pallas as pl
from jax.experimental.pallas import tpu as pltpu


def _vq_body(n_total, d_dim, k_dim, x_ref, emb_ref, qst_ref, enc_ref,
             loss_ref, perp_ref, counts_ref, e2_ref, sse_ref):
    i = pl.program_id(0)
    nb = pl.num_programs(0)
    bn = x_ref.shape[0] * x_ref.shape[1]

    emb = emb_ref[...].reshape(k_dim, d_dim)

    @pl.when(i == 0)
    def _init():
        counts_ref[...] = jnp.zeros_like(counts_ref)
        e2_ref[...] = jnp.sum(emb * emb, axis=1)[None, :]
        sse_ref[0] = 0.0

    x = x_ref[...].reshape(bn, d_dim)
    x2 = jnp.sum(x * x, axis=1, keepdims=True)
    nxe2 = lax.dot_general(-2.0 * x, emb, (((1,), (1,)), ((), ())),
                           preferred_element_type=jnp.float32)
    d = (x2 + nxe2) + e2_ref[...]
    dmin = jnp.min(d, axis=1, keepdims=True)
    kiota = lax.broadcasted_iota(jnp.int32, d.shape, 1)
    kbits = lax.bitcast_convert_type(kiota | jnp.int32(0x3F800000),
                                     jnp.float32)
    sentinel = lax.bitcast_convert_type(jnp.int32(k_dim | 0x3F800000),
                                        jnp.float32)
    idx = jnp.min(jnp.where(d == dmin, kbits, sentinel), axis=1,
                  keepdims=True)
    enc = (kbits == idx).astype(jnp.float32)
    enc_ref[...] = enc.reshape(bn, 1, k_dim)
    q = lax.dot_general(enc, emb, (((1,), (0,)), ((), ())),
                        preferred_element_type=jnp.float32)
    qst_ref[...] = (x + (q - x)).reshape(x_ref.shape)

    counts_ref[...] += jnp.sum(enc, axis=0, keepdims=True)
    sse_ref[0] += jnp.sum(dmin)

    @pl.when(i == nb - 1)
    def _fini():
        loss_ref[0] = sse_ref[0] / (n_total * d_dim)
        avg = counts_ref[...] * (1.0 / n_total)
        perp_ref[0] = jnp.exp(-jnp.sum(avg * jnp.log(avg + 1e-10)))


def kernel(input, embedding):
    b, t, ld = input.shape
    l_dim, k_dim, d_dim = embedding.shape
    n = b * t * (ld // d_dim)
    gb = 2
    bn = (n // b) * gb
    grid = (b // gb,)

    qst, enc, loss, perp = pl.pallas_call(
        functools.partial(_vq_body, n, d_dim, k_dim),
        grid=grid,
        in_specs=[
            pl.BlockSpec((gb, t, ld), lambda i: (i, 0, 0)),
            pl.BlockSpec((l_dim, k_dim, d_dim), lambda i: (0, 0, 0)),
        ],
        out_specs=[
            pl.BlockSpec((gb, t, ld), lambda i: (i, 0, 0)),
            pl.BlockSpec((bn, 1, k_dim), lambda i: (i, 0, 0)),
            pl.BlockSpec(memory_space=pltpu.SMEM),
            pl.BlockSpec(memory_space=pltpu.SMEM),
        ],
        out_shape=[
            jax.ShapeDtypeStruct((b, t, ld), jnp.float32),
            jax.ShapeDtypeStruct((n, l_dim, k_dim), jnp.float32),
            jax.ShapeDtypeStruct((1,), jnp.float32),
            jax.ShapeDtypeStruct((1,), jnp.float32),
        ],
        scratch_shapes=[
            pltpu.VMEM((1, k_dim), jnp.float32),
            pltpu.VMEM((1, k_dim), jnp.float32),
            pltpu.SMEM((1,), jnp.float32),
        ],
    )(input, embedding)

    return qst, enc, loss.reshape(()), perp.reshape(())
